# Initial kernel scaffold; baseline (speedup 1.0000x reference)
import jax
import jax.numpy as jnp
from jax.experimental import pallas as pl


def kernel(x, grad):
    raise NotImplementedError("write your pallas kernel here")


# speedup vs baseline: 43.4268x; 43.4268x over previous
import functools

import jax
import jax.numpy as jnp
from jax.experimental import pallas as pl
from jax.experimental.pallas import tpu as pltpu

P = 0.5
_ROWS = 2048
_COLS = 1024
_N = _ROWS * _COLS
_K = _N // 2
_SCALE = float(1.0 / jnp.float32(1.0 - P + 1e-12))
_CHUNK = 256


def _threshold_kernel(g_ref, thr_ref):
    k = _K

    def count_ge(cand):
        def chunk(j, acc):
            blk = g_ref[0, pl.ds(j * _CHUNK, _CHUNK), :]
            bits = jax.lax.bitcast_convert_type(blk, jnp.int32) & jnp.int32(
                0x7FFFFFFF
            )
            return acc + jnp.sum((bits >= cand).astype(jnp.int32))

        return jax.lax.fori_loop(0, _ROWS // _CHUNK, chunk, jnp.int32(0))

    def body(i, t):
        cand = t | (jnp.int32(1) << (jnp.int32(30) - i))
        cnt = count_ge(cand)
        return jnp.where(cnt >= k, cand, t)

    t_bits = jax.lax.fori_loop(0, 31, body, jnp.int32(0))
    thr = jax.lax.bitcast_convert_type(t_bits, jnp.float32)
    thr_ref[0, 0, :] = jnp.full((128,), thr, dtype=jnp.float32)


def _mask_kernel(thr_ref, x_ref, g_ref, o_ref):
    thr = thr_ref[0, 0, 0]
    mag = jnp.abs(g_ref[0])
    mask = (mag < thr).astype(jnp.float32)
    o_ref[0] = x_ref[0] * mask * _SCALE


@jax.jit
def kernel(x, grad):
    b = x.shape[0]
    thr = pl.pallas_call(
        _threshold_kernel,
        grid=(b,),
        in_specs=[
            pl.BlockSpec((1, _ROWS, _COLS), lambda i: (i, 0, 0)),
        ],
        out_specs=pl.BlockSpec((1, 1, 128), lambda i: (i, 0, 0)),
        out_shape=jax.ShapeDtypeStruct((b, 1, 128), jnp.float32),
    )(grad)

    rows_per_step = 512
    steps = _ROWS // rows_per_step
    out = pl.pallas_call(
        _mask_kernel,
        grid=(b, steps),
        in_specs=[
            pl.BlockSpec((1, 1, 128), lambda i, j: (i, 0, 0)),
            pl.BlockSpec((1, rows_per_step, _COLS), lambda i, j: (i, j, 0)),
            pl.BlockSpec((1, rows_per_step, _COLS), lambda i, j: (i, j, 0)),
        ],
        out_specs=pl.BlockSpec((1, rows_per_step, _COLS), lambda i, j: (i, j, 0)),
        out_shape=jax.ShapeDtypeStruct(x.shape, jnp.float32),
    )(thr, x, grad)
    return out
